# Initial kernel scaffold; baseline (speedup 1.0000x reference)
import functools

import jax
import jax.numpy as jnp
from jax import lax
from jax.experimental import pallas as pl
from jax.experimental.pallas import tpu as pltpu
from jax.experimental.pallas import tpu_sc as plsc

D = 64
L = 128
ROWS_PER_CHUNK = 8
CHUNK = ROWS_PER_CHUNK * L
NW = 32


@functools.lru_cache(maxsize=None)
def _build(num_tokens: int):
    b_per_w = num_tokens // NW
    rows_per_w = b_per_w // L
    chunks = b_per_w // CHUNK

    mesh = plsc.VectorSubcoreMesh(core_axis_name="c", subcore_axis_name="s")

    @functools.partial(
        pl.kernel,
        mesh=mesh,
        out_type=jax.ShapeDtypeStruct((num_tokens, D), jnp.float32),
        scratch_types=[
            pltpu.VMEM((ROWS_PER_CHUNK, L), jnp.int32),
            pltpu.VMEM((CHUNK, D), jnp.float32),
            pltpu.SemaphoreType.DMA,
        ],
    )
    def gather_kernel(idx_hbm, table_hbm, out_hbm, idx_v, rows_v, sem):
        wid = lax.axis_index("s") * 2 + lax.axis_index("c")
        row_base = wid * rows_per_w
        tok_base = wid * b_per_w

        def body(g, carry):
            pltpu.sync_copy(
                idx_hbm.at[pl.ds(row_base + g * ROWS_PER_CHUNK, ROWS_PER_CHUNK)],
                idx_v,
            )
            copies = [
                pltpu.async_copy(
                    table_hbm.at[idx_v.at[j]],
                    rows_v.at[pl.ds(j * L, L)],
                    sem,
                )
                for j in range(ROWS_PER_CHUNK)
            ]
            for c in copies:
                c.wait()
            pltpu.sync_copy(rows_v, out_hbm.at[pl.ds(tok_base + g * CHUNK, CHUNK)])
            return carry

        lax.fori_loop(0, chunks, body, 0)

    return gather_kernel


def kernel(input_, weight):
    b, s = input_.shape
    num_tokens = b * s
    idx2d = input_.astype(jnp.int32).reshape(num_tokens // L, L)
    out = _build(num_tokens)(idx2d, weight)
    return out.reshape(b, s, D)


# speedup vs baseline: 1.8449x; 1.8449x over previous
import functools

import jax
import jax.numpy as jnp
from jax import lax
from jax.experimental import pallas as pl
from jax.experimental.pallas import tpu as pltpu
from jax.experimental.pallas import tpu_sc as plsc

D = 64
L = 128
ROWS_PER_CHUNK = 8
CHUNK = ROWS_PER_CHUNK * L
NW = 32


@functools.lru_cache(maxsize=None)
def _build(num_tokens: int):
    b_per_w = num_tokens // NW
    rows_per_w = b_per_w // L
    chunks = b_per_w // CHUNK

    mesh = plsc.VectorSubcoreMesh(core_axis_name="c", subcore_axis_name="s")

    @functools.partial(
        pl.kernel,
        mesh=mesh,
        out_type=jax.ShapeDtypeStruct((num_tokens, D), jnp.float32),
        scratch_types=[
            pltpu.VMEM((ROWS_PER_CHUNK, L), jnp.int32),
            pltpu.VMEM((CHUNK, D), jnp.float32),
            pltpu.SemaphoreType.DMA,
        ],
        compiler_params=pltpu.CompilerParams(use_tc_tiling_on_sc=False),
    )
    def gather_kernel(idx_hbm, table_hbm, out_hbm, idx_v, rows_v, sem):
        wid = lax.axis_index("s") * 2 + lax.axis_index("c")
        row_base = wid * rows_per_w
        tok_base = wid * b_per_w

        def body(g, carry):
            pltpu.sync_copy(
                idx_hbm.at[pl.ds(row_base + g * ROWS_PER_CHUNK, ROWS_PER_CHUNK)],
                idx_v,
            )
            copies = [
                pltpu.async_copy(
                    table_hbm.at[idx_v.at[j]],
                    rows_v.at[pl.ds(j * L, L)],
                    sem,
                )
                for j in range(ROWS_PER_CHUNK)
            ]
            for c in copies:
                c.wait()
            pltpu.sync_copy(rows_v, out_hbm.at[pl.ds(tok_base + g * CHUNK, CHUNK)])
            return carry

        lax.fori_loop(0, chunks, body, 0)

    return gather_kernel


def kernel(input_, weight):
    b, s = input_.shape
    num_tokens = b * s
    idx2d = input_.astype(jnp.int32).reshape(num_tokens // L, L)
    out = _build(num_tokens)(idx2d, weight)
    return out.reshape(b, s, D)


# speedup vs baseline: 1.8745x; 1.0160x over previous
import functools

import jax
import jax.numpy as jnp
from jax import lax
from jax.experimental import pallas as pl
from jax.experimental.pallas import tpu as pltpu
from jax.experimental.pallas import tpu_sc as plsc

D = 64
L = 128
ROWS_PER_CHUNK = 5
CHUNK = ROWS_PER_CHUNK * L
NW = 32


@functools.lru_cache(maxsize=None)
def _build(num_tokens: int):
    b_per_w = num_tokens // NW
    rows_per_w = b_per_w // L
    chunks = b_per_w // CHUNK

    mesh = plsc.VectorSubcoreMesh(core_axis_name="c", subcore_axis_name="s")

    @functools.partial(
        pl.kernel,
        mesh=mesh,
        out_type=jax.ShapeDtypeStruct((num_tokens, D), jnp.float32),
        scratch_types=[
            pltpu.VMEM((rows_per_w, L), jnp.int32),
            pltpu.VMEM((CHUNK, D), jnp.float32),
            pltpu.VMEM((CHUNK, D), jnp.float32),
            pltpu.SemaphoreType.DMA,
            pltpu.SemaphoreType.DMA,
            pltpu.SemaphoreType.DMA,
            pltpu.SemaphoreType.DMA,
        ],
        compiler_params=pltpu.CompilerParams(use_tc_tiling_on_sc=False),
    )
    def gather_kernel(idx_hbm, table_hbm, out_hbm, idx_v, rows0, rows1,
                      gsem0, gsem1, ssem0, ssem1):
        wid = lax.axis_index("s") * 2 + lax.axis_index("c")
        row_base = wid * rows_per_w
        tok_base = wid * b_per_w
        bufs = (rows0, rows1)
        gsems = (gsem0, gsem1)
        ssems = (ssem0, ssem1)

        pltpu.sync_copy(idx_hbm.at[pl.ds(row_base, rows_per_w)], idx_v)

        def fire_gathers(g, b):
            for j in range(ROWS_PER_CHUNK):
                pltpu.async_copy(
                    table_hbm.at[idx_v.at[g * ROWS_PER_CHUNK + j]],
                    bufs[b].at[pl.ds(j * L, L)],
                    gsems[b],
                )

        def drain_gathers(b):
            for j in range(ROWS_PER_CHUNK):
                pltpu.make_async_copy(
                    table_hbm.at[idx_v.at[0]],
                    bufs[b].at[pl.ds(j * L, L)],
                    gsems[b],
                ).wait()

        def store_chunk(g, b):
            pltpu.async_copy(
                bufs[b], out_hbm.at[pl.ds(tok_base + g * CHUNK, CHUNK)], ssems[b]
            )

        def wait_store(b):
            pltpu.make_async_copy(
                bufs[b], out_hbm.at[pl.ds(tok_base, CHUNK)], ssems[b]
            ).wait()

        fire_gathers(0, 0)

        def body(i, carry):
            for b in range(2):
                g = 2 * i + b
                nb = 1 - b
                @pl.when(g >= 1)
                def _():
                    wait_store(nb)

                @pl.when(g + 1 < chunks)
                def _():
                    fire_gathers(g + 1, nb)

                drain_gathers(b)
                store_chunk(g, b)
            return carry

        lax.fori_loop(0, chunks // 2, body, 0)

        wait_store((chunks - 1) % 2)

    return gather_kernel


def kernel(input_, weight):
    b, s = input_.shape
    num_tokens = b * s
    idx2d = input_.astype(jnp.int32).reshape(num_tokens // L, L)
    out = _build(num_tokens)(idx2d, weight)
    return out.reshape(b, s, D)
